# speedup vs baseline: 31.5087x; 31.5087x over previous
import jax
import jax.numpy as jnp
from jax.experimental import pallas as pl


D = 8
F = 128
TB = 1024


def _body(x_ref, w1_ref, b1_ref, w2_ref, b2_ref, dom_ref, out_ref):
    h = jnp.maximum(
        jnp.dot(x_ref[...], w1_ref[...], preferred_element_type=jnp.float32)
        + b1_ref[...],
        0.0,
    )
    logits = (
        jnp.dot(h, w2_ref[...], preferred_element_type=jnp.float32) + b2_ref[...]
    )
    m = jnp.max(logits, axis=-1, keepdims=True)
    e = jnp.exp(logits - m)
    gate = e / jnp.sum(e, axis=-1, keepdims=True)

    acc = gate[:, 0:1] * dom_ref[0]
    for d in range(1, D):
        acc += gate[:, d : d + 1] * dom_ref[d]
    out_ref[...] = acc


@jax.jit
def _run(domain_outputs, x, W1, b1, W2, b2):
    B = x.shape[0]
    din = x.shape[1]
    H = W1.shape[1]
    grid = (B // TB,)
    return pl.pallas_call(
        _body,
        grid=grid,
        in_specs=[
            pl.BlockSpec((TB, din), lambda i: (i, 0)),
            pl.BlockSpec((din, H), lambda i: (0, 0)),
            pl.BlockSpec((1, H), lambda i: (0, 0)),
            pl.BlockSpec((H, D), lambda i: (0, 0)),
            pl.BlockSpec((1, D), lambda i: (0, 0)),
            pl.BlockSpec((D, TB, F), lambda i: (0, i, 0)),
        ],
        out_specs=pl.BlockSpec((TB, F), lambda i: (i, 0)),
        out_shape=jax.ShapeDtypeStruct((B, F), jnp.float32),
    )(x, W1, b1.reshape(1, H), W2, b2.reshape(1, D), domain_outputs)


def kernel(domain_outputs, x, domain_ids, W1, b1, W2, b2):
    del domain_ids
    return _run(domain_outputs, x, W1, b1, W2, b2)


# speedup vs baseline: 33.2048x; 1.0538x over previous
import jax
import jax.numpy as jnp
from jax.experimental import pallas as pl


D = 8
F = 128
TB = 2048


def _body(x_ref, w1_ref, b1_ref, w2_ref, b2_ref, dom_ref, out_ref):
    h = jnp.maximum(
        jnp.dot(x_ref[...], w1_ref[...], preferred_element_type=jnp.float32)
        + b1_ref[...],
        0.0,
    )
    logits = (
        jnp.dot(h, w2_ref[...], preferred_element_type=jnp.float32) + b2_ref[...]
    )
    m = jnp.max(logits, axis=-1, keepdims=True)
    e = jnp.exp(logits - m)
    gate = e / jnp.sum(e, axis=-1, keepdims=True)

    acc = gate[:, 0:1] * dom_ref[0]
    for d in range(1, D):
        acc += gate[:, d : d + 1] * dom_ref[d]
    out_ref[...] = acc


@jax.jit
def _run(domain_outputs, x, W1, b1, W2, b2):
    B = x.shape[0]
    din = x.shape[1]
    H = W1.shape[1]
    grid = (B // TB,)
    return pl.pallas_call(
        _body,
        grid=grid,
        in_specs=[
            pl.BlockSpec((TB, din), lambda i: (i, 0)),
            pl.BlockSpec((din, H), lambda i: (0, 0)),
            pl.BlockSpec((1, H), lambda i: (0, 0)),
            pl.BlockSpec((H, D), lambda i: (0, 0)),
            pl.BlockSpec((1, D), lambda i: (0, 0)),
            pl.BlockSpec((D, TB, F), lambda i: (0, i, 0)),
        ],
        out_specs=pl.BlockSpec((TB, F), lambda i: (i, 0)),
        out_shape=jax.ShapeDtypeStruct((B, F), jnp.float32),
    )(x, W1, b1.reshape(1, H), W2, b2.reshape(1, D), domain_outputs)


def kernel(domain_outputs, x, domain_ids, W1, b1, W2, b2):
    del domain_ids
    return _run(domain_outputs, x, W1, b1, W2, b2)


# speedup vs baseline: 52.6086x; 1.5844x over previous
import jax
import jax.numpy as jnp
from jax.experimental import pallas as pl


D = 8
F = 128
TB = 2048


def _body(x_ref, w1_ref, b1_ref, w2_ref, b2_ref, exp_ref, dom_ref, out_ref):
    h = jnp.maximum(
        jnp.dot(x_ref[...], w1_ref[...], preferred_element_type=jnp.float32)
        + b1_ref[...],
        0.0,
    )
    logits = (
        jnp.dot(h, w2_ref[...], preferred_element_type=jnp.float32) + b2_ref[...]
    )
    m = jnp.max(logits, axis=-1, keepdims=True)
    e = jnp.exp(logits - m)
    gate = e / jnp.sum(e, axis=-1, keepdims=True)

    bcast = jnp.dot(gate, exp_ref[...], preferred_element_type=jnp.float32)
    acc = bcast[:, 0:F] * dom_ref[0]
    for d in range(1, D):
        acc += bcast[:, d * F : (d + 1) * F] * dom_ref[d]
    out_ref[...] = acc


@jax.jit
def _run(domain_outputs, x, W1, b1, W2, b2):
    B = x.shape[0]
    din = x.shape[1]
    H = W1.shape[1]
    grid = (B // TB,)
    expander = jnp.kron(jnp.eye(D, dtype=jnp.float32), jnp.ones((1, F), jnp.float32))
    return pl.pallas_call(
        _body,
        grid=grid,
        in_specs=[
            pl.BlockSpec((TB, din), lambda i: (i, 0)),
            pl.BlockSpec((din, H), lambda i: (0, 0)),
            pl.BlockSpec((1, H), lambda i: (0, 0)),
            pl.BlockSpec((H, D), lambda i: (0, 0)),
            pl.BlockSpec((1, D), lambda i: (0, 0)),
            pl.BlockSpec((D, D * F), lambda i: (0, 0)),
            pl.BlockSpec((D, TB, F), lambda i: (0, i, 0)),
        ],
        out_specs=pl.BlockSpec((TB, F), lambda i: (i, 0)),
        out_shape=jax.ShapeDtypeStruct((B, F), jnp.float32),
    )(x, W1, b1.reshape(1, H), W2, b2.reshape(1, D), expander, domain_outputs)


def kernel(domain_outputs, x, domain_ids, W1, b1, W2, b2):
    del domain_ids
    return _run(domain_outputs, x, W1, b1, W2, b2)
